# baseline (speedup 1.0000x reference)
import functools

import jax
import jax.numpy as jnp
import numpy as np
from jax.experimental import pallas as pl
from jax.experimental.pallas import tpu as pltpu

_ALPHA = 0.98
_DELTA = 2.0
_R = 0.5
_S = 0.025
_EPS = 1e-6
_A = 1.0 - _S


def _pcen_kernel(x_ref, l_ref, apow_ref, o_ref, carry_ref, *, t_total, wt):
    t = pl.program_id(1)
    bb, _, c = x_ref.shape
    n = bb * c
    xw = jnp.concatenate([x_ref[b] for b in range(bb)], axis=1)
    sub = jax.lax.broadcasted_iota(jnp.int32, (wt, n), 0)
    xb = jnp.where(sub < (t_total - t * wt), xw, 0.0)

    @pl.when(t == 0)
    def _():
        carry_ref[...] = xb[0:1, :]

    ap = jnp.concatenate([apow_ref[...]] * bb, axis=1)
    cb = jnp.broadcast_to(carry_ref[...], (wt, n))
    sm = (
        jnp.dot(l_ref[...], xb.astype(jnp.bfloat16), preferred_element_type=jnp.float32)
        + ap * cb
    )
    carry_ref[...] = sm[wt - 1 : wt, :]

    inv_pow = jnp.exp2(jnp.log2(sm + _EPS) * (-_ALPHA))
    u = xb * inv_pow + _DELTA
    out = jax.lax.rsqrt(u) * u - np.float32(np.sqrt(_DELTA))
    for b in range(bb):
        o_ref[b] = out[:, b * c : (b + 1) * c]


def _build_consts(wt, c):
    j = np.arange(wt)[:, None].astype(np.float64)
    k = np.arange(wt)[None, :].astype(np.float64)
    l_mat = np.where(j >= k, _S * _A ** (j - k), 0.0).astype(np.float32)
    apow = np.broadcast_to(
        (_A ** (np.arange(wt, dtype=np.float64) + 1.0)).astype(np.float32)[:, None],
        (wt, c),
    ).copy()
    return l_mat, apow


@jax.jit
def kernel(x):
    b, c, t_total = x.shape
    xt = jnp.transpose(x, (0, 2, 1))

    wt = 256
    bb = 64
    n_chunks = pl.cdiv(t_total, wt)
    n_b_tiles = pl.cdiv(b, bb)

    l_mat, apow = _build_consts(wt, c)

    out = pl.pallas_call(
        functools.partial(_pcen_kernel, t_total=t_total, wt=wt),
        out_shape=jax.ShapeDtypeStruct((b, t_total, c), jnp.float32),
        grid=(n_b_tiles, n_chunks),
        in_specs=[
            pl.BlockSpec((bb, wt, c), lambda i, t: (i, t, 0)),
            pl.BlockSpec((wt, wt), lambda i, t: (0, 0)),
            pl.BlockSpec((wt, c), lambda i, t: (0, 0)),
        ],
        out_specs=pl.BlockSpec((bb, wt, c), lambda i, t: (i, t, 0)),
        scratch_shapes=[pltpu.VMEM((1, bb * c), jnp.float32)],
        compiler_params=pltpu.CompilerParams(
            dimension_semantics=("parallel", "arbitrary"),
        ),
        name="pcen",
    )(xt, jnp.asarray(l_mat, dtype=jnp.bfloat16), jnp.asarray(apow))

    return jnp.transpose(out, (0, 2, 1))


# speedup vs baseline: 1.0271x; 1.0271x over previous
import functools

import jax
import jax.numpy as jnp
import numpy as np
from jax.experimental import pallas as pl
from jax.experimental.pallas import tpu as pltpu

_ALPHA = 0.98
_DELTA = 2.0
_R = 0.5
_S = 0.025
_EPS = 1e-6
_A = 1.0 - _S


def _pcen_kernel(x_ref, l_ref, o_ref, carry_ref, *, wt):
    t = pl.program_id(1)
    bb, _, c = x_ref.shape
    n = bb * c
    xw = jnp.concatenate([x_ref[b] for b in range(bb)], axis=1)

    @pl.when(t == 0)
    def _():
        row_i = jax.lax.broadcasted_iota(jnp.int32, (8, n), 0)
        x0 = jnp.broadcast_to(xw[0:1, :] + np.float32(_EPS), (8, n))
        carry_ref[...] = jnp.where(row_i == 0, x0, jnp.where(row_i == 1, 1.0, 0.0))

    xaug = jnp.concatenate([xw, carry_ref[...]], axis=0)
    sm_eps = jnp.dot(
        l_ref[...], xaug.astype(jnp.bfloat16), preferred_element_type=jnp.float32
    )
    carry_ref[0:1, :] = sm_eps[wt - 1 : wt, :]

    inv_pow = jnp.exp2(jnp.log2(sm_eps) * (-_ALPHA))
    u = xw * inv_pow + _DELTA
    out = jax.lax.rsqrt(u) * u - np.float32(np.sqrt(_DELTA))
    for b in range(bb):
        o_ref[b] = out[:, b * c : (b + 1) * c]


def _build_l_aug(wt):
    j = np.arange(wt)[:, None].astype(np.float64)
    k = np.arange(wt)[None, :].astype(np.float64)
    l_aug = np.zeros((wt, wt + 8), dtype=np.float64)
    l_aug[:, :wt] = np.where(j >= k, _S * _A ** (j - k), 0.0)
    aj = _A ** (np.arange(wt, dtype=np.float64) + 1.0)
    l_aug[:, wt] = aj
    l_aug[:, wt + 1] = _EPS * (1.0 - aj)
    return l_aug.astype(np.float32)


@jax.jit
def kernel(x):
    b, c, t_total = x.shape
    xt = jnp.transpose(x, (0, 2, 1))

    wt = 160
    bb = b if b < 64 else 64
    n_chunks = pl.cdiv(t_total, wt)
    n_b_tiles = pl.cdiv(b, bb)

    l_aug = _build_l_aug(wt)

    out = pl.pallas_call(
        functools.partial(_pcen_kernel, wt=wt),
        out_shape=jax.ShapeDtypeStruct((b, t_total, c), jnp.float32),
        grid=(n_b_tiles, n_chunks),
        in_specs=[
            pl.BlockSpec((bb, wt, c), lambda i, t: (i, t, 0)),
            pl.BlockSpec((wt, wt + 8), lambda i, t: (0, 0)),
        ],
        out_specs=pl.BlockSpec((bb, wt, c), lambda i, t: (i, t, 0)),
        scratch_shapes=[pltpu.VMEM((8, bb * c), jnp.float32)],
        compiler_params=pltpu.CompilerParams(
            dimension_semantics=("parallel", "arbitrary"),
        ),
        name="pcen",
    )(xt, jnp.asarray(l_aug, dtype=jnp.bfloat16))

    return jnp.transpose(out, (0, 2, 1))


# speedup vs baseline: 1.0527x; 1.0249x over previous
import functools

import jax
import jax.numpy as jnp
import numpy as np
from jax.experimental import pallas as pl
from jax.experimental.pallas import tpu as pltpu

_ALPHA = 0.98
_DELTA = 2.0
_R = 0.5
_S = 0.025
_EPS = 1e-6
_A = 1.0 - _S


def _pcen_kernel(x_ref, l_ref, o_ref, carry_ref, *, wt):
    t = pl.program_id(1)
    bb, _, c = x_ref.shape
    n = bb * c
    xw = jnp.concatenate([x_ref[b] for b in range(bb)], axis=1)

    @pl.when(t == 0)
    def _():
        row_i = jax.lax.broadcasted_iota(jnp.int32, (8, n), 0)
        x0 = jnp.broadcast_to(xw[0:1, :] + np.float32(_EPS), (8, n))
        carry_ref[...] = jnp.where(row_i == 0, x0, jnp.where(row_i == 1, 1.0, 0.0))

    xaug = jnp.concatenate([xw, carry_ref[...]], axis=0)
    sm_eps = jnp.dot(
        l_ref[...], xaug.astype(jnp.bfloat16), preferred_element_type=jnp.float32
    )
    carry_ref[0:1, :] = sm_eps[wt - 1 : wt, :]

    inv_pow = jnp.exp2(jnp.log2(sm_eps) * (-_ALPHA))
    u = xw * inv_pow + _DELTA
    out = jax.lax.rsqrt(u) * u - np.float32(np.sqrt(_DELTA))
    for b in range(bb):
        o_ref[b] = out[:, b * c : (b + 1) * c]


def _build_l_aug(wt):
    j = np.arange(wt)[:, None].astype(np.float64)
    k = np.arange(wt)[None, :].astype(np.float64)
    l_aug = np.zeros((wt, wt + 8), dtype=np.float64)
    l_aug[:, :wt] = np.where(j >= k, _S * _A ** (j - k), 0.0)
    aj = _A ** (np.arange(wt, dtype=np.float64) + 1.0)
    l_aug[:, wt] = aj
    l_aug[:, wt + 1] = _EPS * (1.0 - aj)
    return l_aug.astype(np.float32)


@jax.jit
def kernel(x):
    b, c, t_total = x.shape
    xt = jnp.transpose(x, (0, 2, 1))

    wt = 200
    bb = b if b < 64 else 64
    n_chunks = pl.cdiv(t_total, wt)
    n_b_tiles = pl.cdiv(b, bb)

    l_aug = _build_l_aug(wt)

    out = pl.pallas_call(
        functools.partial(_pcen_kernel, wt=wt),
        out_shape=jax.ShapeDtypeStruct((b, t_total, c), jnp.float32),
        grid=(n_b_tiles, n_chunks),
        in_specs=[
            pl.BlockSpec((bb, wt, c), lambda i, t: (i, t, 0)),
            pl.BlockSpec((wt, wt + 8), lambda i, t: (0, 0)),
        ],
        out_specs=pl.BlockSpec((bb, wt, c), lambda i, t: (i, t, 0)),
        scratch_shapes=[pltpu.VMEM((8, bb * c), jnp.float32)],
        compiler_params=pltpu.CompilerParams(
            dimension_semantics=("parallel", "arbitrary"),
        ),
        name="pcen",
    )(xt, jnp.asarray(l_aug, dtype=jnp.bfloat16))

    return jnp.transpose(out, (0, 2, 1))
